# Initial kernel scaffold; baseline (speedup 1.0000x reference)
import jax
import jax.numpy as jnp
from jax.experimental import pallas as pl


def kernel(x, edge_index, batch, W1, b1, g1, be1, W2, b2, g2, be2, Wo, bo):
    raise NotImplementedError("write your pallas kernel here")


# speedup vs baseline: 9.4445x; 9.4445x over previous
import functools

import jax
import jax.numpy as jnp
from jax import lax
from jax.experimental import pallas as pl
from jax.experimental.pallas import tpu as pltpu
from jax.experimental.pallas import tpu_sc as plsc

_N = 10000
_E = 320000
_D = 128
_G = 64

_NC = 2
_NS = 16
_CHUNK = 128
_TILES = _NC * _NS
_CHUNKS_PER_TILE = -(-_E // (_CHUNK * _TILES))
_E_PAD = _CHUNK * _TILES * _CHUNKS_PER_TILE
_EDGES_PER_TILE = _CHUNK * _CHUNKS_PER_TILE
_EDGES_PER_SC = _EDGES_PER_TILE * _NS
_ACC_ROWS = 10240
_ROWS_PER_TILE = _ACC_ROWS // _NS

_mesh = plsc.VectorSubcoreMesh(core_axis_name="c", subcore_axis_name="s")


@functools.partial(
    pl.kernel,
    out_type=jax.ShapeDtypeStruct((_TILES, _ACC_ROWS), jnp.float32),
    mesh=_mesh,
    scratch_types=[
        pltpu.VMEM((_CHUNK,), jnp.int32),
        pltpu.VMEM((_ACC_ROWS,), jnp.float32),
    ],
    compiler_params=pltpu.CompilerParams(needs_layout_passes=False),
)
def _sc_degree(dst_hbm, zeros_hbm, out_hbm, idx_v, acc_v):
    c = lax.axis_index("c")
    s = lax.axis_index("s")
    wid = c * _NS + s
    pltpu.sync_copy(zeros_hbm, acc_v)
    ones = jnp.ones((16,), jnp.float32)

    base = c * _EDGES_PER_SC + s * _EDGES_PER_TILE

    def body(g, carry):
        off = base + g * _CHUNK
        pltpu.sync_copy(dst_hbm.at[pl.ds(off, _CHUNK)], idx_v)
        for j in range(_CHUNK // 16):
            idx = idx_v[pl.ds(j * 16, 16)]
            plsc.addupdate_scatter(acc_v, [idx], ones)
        return carry

    lax.fori_loop(0, _CHUNKS_PER_TILE, body, 0)
    pltpu.sync_copy(acc_v, out_hbm.at[wid])


@functools.partial(
    pl.kernel,
    out_type=jax.ShapeDtypeStruct((_NC, _ACC_ROWS, _D), jnp.float32),
    mesh=_mesh,
    scratch_types=[
        pltpu.VMEM((_CHUNK,), jnp.int32),
        pltpu.VMEM((_CHUNK,), jnp.int32),
        pltpu.VMEM((_CHUNK, _D), jnp.float32),
        pltpu.VMEM_SHARED((_ACC_ROWS, _D), jnp.float32),
        pltpu.SemaphoreType.DMA,
    ],
)
def _sc_aggregate(hs_hbm, src_hbm, dst_hbm, zeros_hbm, out_hbm,
                  src_v, dst_v, rows_v, acc_s, sem):
    c = lax.axis_index("c")
    s = lax.axis_index("s")
    row0 = s * _ROWS_PER_TILE
    pltpu.sync_copy(zeros_hbm, acc_s.at[pl.ds(row0, _ROWS_PER_TILE)])
    plsc.subcore_barrier()

    base = c * _EDGES_PER_SC + s * _EDGES_PER_TILE

    def body(g, carry):
        off = base + g * _CHUNK
        pltpu.sync_copy(src_hbm.at[pl.ds(off, _CHUNK)], src_v)
        pltpu.sync_copy(dst_hbm.at[pl.ds(off, _CHUNK)], dst_v)
        pltpu.async_copy(hs_hbm.at[src_v], rows_v, sem).wait()
        pltpu.sync_copy(rows_v, acc_s.at[dst_v], add=True)
        return carry

    lax.fori_loop(0, _CHUNKS_PER_TILE, body, 0)
    plsc.subcore_barrier()
    pltpu.sync_copy(
        acc_s.at[pl.ds(row0, _ROWS_PER_TILE)],
        out_hbm.at[c, pl.ds(row0, _ROWS_PER_TILE)],
    )


def _mm(a, b_t):
    return lax.dot_general(a, b_t, (((1,), (1,)), ((), ())),
                           preferred_element_type=jnp.float32)


def _tc1_body(x_ref, w1_ref, degp_ref, hs1_ref, dinv_ref):
    deg = jnp.sum(degp_ref[:, : _N], axis=0) + 1.0
    dinv = lax.rsqrt(deg)
    h1 = _mm(x_ref[...], w1_ref[...])
    hs1_ref[...] = h1 * dinv[:, None]
    dinv_ref[...] = dinv


def _tc2_body(msgp_ref, hs1_ref, dinv_ref, b1_ref, g1_ref, be1_ref, w2_ref,
              hs2_ref):
    dinv = dinv_ref[...]
    msg = msgp_ref[0, : _N, :] + msgp_ref[1, : _N, :]
    t = dinv[:, None] * (msg + hs1_ref[...]) + b1_ref[...][None, :]
    mu = jnp.mean(t, axis=0)
    var = jnp.mean((t - mu[None, :]) ** 2, axis=0)
    y = (t - mu[None, :]) * lax.rsqrt(var + 1e-5)[None, :] * g1_ref[...][None, :]
    y = jnp.maximum(y + be1_ref[...][None, :], 0.0)
    h2 = _mm(y, w2_ref[...])
    hs2_ref[...] = h2 * dinv[:, None]


def _tc3_body(msgp_ref, hs2_ref, dinv_ref, b2_ref, g2_ref, be2_ref,
              batch_ref, wo_ref, bo_ref, out_ref):
    dinv = dinv_ref[...]
    msg = msgp_ref[0, : _N, :] + msgp_ref[1, : _N, :]
    t = dinv[:, None] * (msg + hs2_ref[...]) + b2_ref[...][None, :]
    mu = jnp.mean(t, axis=0)
    var = jnp.mean((t - mu[None, :]) ** 2, axis=0)
    y = (t - mu[None, :]) * lax.rsqrt(var + 1e-5)[None, :] * g2_ref[...][None, :]
    y = jnp.maximum(y + be2_ref[...][None, :], 0.0)

    gids = lax.broadcasted_iota(jnp.int32, (_N, _G), 1)
    seg = (batch_ref[...][:, None] == gids).astype(jnp.float32)
    sums = lax.dot_general(seg, y, (((0,), (0,)), ((), ())),
                           preferred_element_type=jnp.float32)
    cnt = jnp.sum(seg, axis=0)
    mean = sums / jnp.maximum(cnt, 1.0)[:, None]
    out_ref[...] = _mm(mean, wo_ref[...]) + bo_ref[...][None, :]


def kernel(x, edge_index, batch, W1, b1, g1, be1, W2, b2, g2, be2, Wo, bo):
    src = edge_index[0].astype(jnp.int32)
    dst = edge_index[1].astype(jnp.int32)
    pad = _E_PAD - _E
    srcp = jnp.concatenate([src, jnp.zeros((pad,), jnp.int32)])
    dstp = jnp.concatenate([dst, jnp.full((pad,), _N, jnp.int32)])

    zeros1d = jnp.zeros((_ACC_ROWS,), jnp.float32)
    zerosD = jnp.zeros((_ROWS_PER_TILE, _D), jnp.float32)

    degp = _sc_degree(dstp, zeros1d)

    hs1, dinv = pl.pallas_call(
        _tc1_body,
        out_shape=(
            jax.ShapeDtypeStruct((_N, _D), jnp.float32),
            jax.ShapeDtypeStruct((_N,), jnp.float32),
        ),
    )(x, W1, degp)

    msg1 = _sc_aggregate(hs1, srcp, dstp, zerosD)

    hs2 = pl.pallas_call(
        _tc2_body,
        out_shape=jax.ShapeDtypeStruct((_N, _D), jnp.float32),
    )(msg1, hs1, dinv, b1, g1, be1, W2)

    msg2 = _sc_aggregate(hs2, srcp, dstp, zerosD)

    out = pl.pallas_call(
        _tc3_body,
        out_shape=jax.ShapeDtypeStruct((_G, _D), jnp.float32),
    )(msg2, hs2, dinv, b2, g2, be2, batch.astype(jnp.int32), Wo, bo)
    return out
